# baseline (speedup 1.0000x reference)
import functools

import jax
import jax.numpy as jnp
from jax import lax
from jax.experimental import pallas as pl
from jax.experimental.pallas import tpu as pltpu
from jax.experimental.pallas import tpu_sc as plsc

BATCH = 16384
EMBED = 64
LANES = 16
CHUNK = 128


def _gather_body(bpw, nchunks, nc,
                 cidx_hbm, ctab_hbm, rows_hbm,
                 cidx_v, crows_v, sem):
    wid = lax.axis_index("s") * nc + lax.axis_index("c")

    pltpu.sync_copy(cidx_hbm.at[wid], cidx_v)
    copies = []
    for j in range(nchunks):
        row = pl.ds(j * CHUNK, CHUNK)
        copies.append(pltpu.async_copy(ctab_hbm.at[cidx_v.at[j]],
                                       crows_v.at[row], sem))
    for c in copies:
        c.wait()
    pltpu.sync_copy(crows_v, rows_hbm.at[pl.ds(wid * bpw, bpw)])


def _dot_body(bpw, nchunks, nc,
              cidx_hbm, aidx_hbm, atab_hbm, crows_hbm, cbias_hbm, abias_hbm,
              out_hbm,
              cidx_v, aidx_v, crows_v, arows_v, cb_v, ab_v, out_v, tbuf, sem):
    wid = lax.axis_index("s") * nc + lax.axis_index("c")

    pltpu.sync_copy(cidx_hbm.at[wid], cidx_v)
    pltpu.sync_copy(aidx_hbm.at[wid], aidx_v)

    copies = []
    for j in range(nchunks):
        row = pl.ds(j * CHUNK, CHUNK)
        copies.append(pltpu.async_copy(atab_hbm.at[aidx_v.at[j]],
                                       arows_v.at[row], sem))
        copies.append(pltpu.async_copy(cbias_hbm.at[cidx_v.at[j]],
                                       cb_v.at[row], sem))
        copies.append(pltpu.async_copy(abias_hbm.at[aidx_v.at[j]],
                                       ab_v.at[row], sem))
    for c in copies:
        c.wait()

    lane_ids = lax.iota(jnp.int32, LANES)
    half = bpw // 2

    for h in range(2):
        pltpu.sync_copy(crows_hbm.at[pl.ds(wid * bpw + h * half, half)],
                        crows_v)

        def body(g, carry):
            base_i = h * half + g * LANES
            for t in range(LANES):
                i = base_i + t
                acc = jnp.zeros((LANES,), jnp.float32)
                for k in range(EMBED // LANES):
                    col = pl.ds(k * LANES, LANES)
                    acc = acc + crows_v[g * LANES + t, col] * arows_v[i, col]
                plsc.store_scatter(tbuf, [lane_ids * LANES + t], acc)
            sums = jnp.zeros((LANES,), jnp.float32)
            for l in range(LANES):
                sums = sums + tbuf[pl.ds(l * LANES, LANES)]
            grp = pl.ds(base_i, LANES)
            out_v[grp] = sums + cb_v[grp] + ab_v[grp]
            return carry

        lax.fori_loop(0, half // LANES, body, 0)

    pltpu.sync_copy(out_v, out_hbm.at[pl.ds(wid * bpw, bpw)])


def kernel(customer_idx, article_idx, customer_emb_table, article_emb_table,
           customer_bias_table, article_bias_table):
    info = plsc.get_sparse_core_info()
    nc, ns = 1, info.num_subcores
    nw = nc * ns
    bpw = BATCH // nw
    nchunks = bpw // CHUNK

    cidx = customer_idx.astype(jnp.int32).reshape(nw, nchunks, CHUNK)
    aidx = article_idx.astype(jnp.int32).reshape(nw, nchunks, CHUNK)
    cbias = customer_bias_table.reshape(-1)
    abias = article_bias_table.reshape(-1)

    mesh = plsc.VectorSubcoreMesh(core_axis_name="c", subcore_axis_name="s",
                                  num_cores=1)
    params = pltpu.CompilerParams(needs_layout_passes=False,
                                  use_tc_tiling_on_sc=False)

    k1 = pl.kernel(
        functools.partial(_gather_body, bpw, nchunks, nc),
        out_type=jax.ShapeDtypeStruct((BATCH, EMBED), jnp.float32),
        mesh=mesh,
        compiler_params=params,
        scratch_types=[
            pltpu.VMEM((nchunks, CHUNK), jnp.int32),
            pltpu.VMEM((bpw, EMBED), jnp.float32),
            pltpu.SemaphoreType.DMA,
        ],
    )
    crows = k1(cidx, customer_emb_table)

    k2 = pl.kernel(
        functools.partial(_dot_body, bpw, nchunks, nc),
        out_type=jax.ShapeDtypeStruct((BATCH,), jnp.float32),
        mesh=mesh,
        compiler_params=params,
        scratch_types=[
            pltpu.VMEM((nchunks, CHUNK), jnp.int32),
            pltpu.VMEM((nchunks, CHUNK), jnp.int32),
            pltpu.VMEM((bpw // 2, EMBED), jnp.float32),
            pltpu.VMEM((bpw, EMBED), jnp.float32),
            pltpu.VMEM((bpw,), jnp.float32),
            pltpu.VMEM((bpw,), jnp.float32),
            pltpu.VMEM((bpw,), jnp.float32),
            pltpu.VMEM((LANES * LANES,), jnp.float32),
            pltpu.SemaphoreType.DMA,
        ],
    )
    return k2(cidx, aidx, article_emb_table, crows, cbias, abias)


# speedup vs baseline: 1.0879x; 1.0879x over previous
import functools

import jax
import jax.numpy as jnp
from jax import lax
from jax.experimental import pallas as pl
from jax.experimental.pallas import tpu as pltpu
from jax.experimental.pallas import tpu_sc as plsc

BATCH = 16384
EMBED = 64
LANES = 16
CHUNK = 128


def _sc_body(bpw, nchunks, nc,
             cidx_hbm, aidx_hbm, ctab_hbm, atab_hbm, cbias_hbm, abias_hbm,
             out_hbm,
             cidx_v, aidx_v, crows_v, arows_v, cb_v, ab_v, out_v, tbuf, sem):
    wid = lax.axis_index("s") * nc + lax.axis_index("c")

    pltpu.sync_copy(cidx_hbm.at[wid], cidx_v)
    pltpu.sync_copy(aidx_hbm.at[wid], aidx_v)

    copies = []
    for j in range(nchunks):
        row = pl.ds(j * CHUNK, CHUNK)
        copies.append(pltpu.async_copy(ctab_hbm.at[cidx_v.at[j]],
                                       crows_v.at[row], sem))
        copies.append(pltpu.async_copy(atab_hbm.at[aidx_v.at[j]],
                                       arows_v.at[row], sem))
        copies.append(pltpu.async_copy(cbias_hbm.at[cidx_v.at[j]],
                                       cb_v.at[row], sem))
        copies.append(pltpu.async_copy(abias_hbm.at[aidx_v.at[j]],
                                       ab_v.at[row], sem))
    for c in copies:
        c.wait()

    lane_ids = lax.iota(jnp.int32, LANES)

    def body(g, carry):
        base_i = g * LANES
        for t in range(LANES):
            i = base_i + t
            acc = jnp.zeros((LANES,), jnp.float32)
            for k in range(EMBED // LANES):
                col = pl.ds(k * LANES, LANES)
                acc = acc + crows_v[i, col] * arows_v[i, col]
            plsc.store_scatter(tbuf, [lane_ids * LANES + t], acc)
        sums = jnp.zeros((LANES,), jnp.float32)
        for l in range(LANES):
            sums = sums + tbuf[pl.ds(l * LANES, LANES)]
        grp = pl.ds(base_i, LANES)
        out_v[grp] = sums + cb_v[grp] + ab_v[grp]
        return carry

    lax.fori_loop(0, bpw // LANES, body, 0)

    pltpu.sync_copy(out_v, out_hbm.at[pl.ds(wid * bpw, bpw)])


def kernel(customer_idx, article_idx, customer_emb_table, article_emb_table,
           customer_bias_table, article_bias_table):
    info = plsc.get_sparse_core_info()
    nc, ns = info.num_cores, info.num_subcores
    nw = nc * ns
    bpw = BATCH // nw
    nchunks = bpw // CHUNK

    cidx = customer_idx.astype(jnp.int32).reshape(nw, nchunks, CHUNK)
    aidx = article_idx.astype(jnp.int32).reshape(nw, nchunks, CHUNK)
    cbias = customer_bias_table.reshape(-1)
    abias = article_bias_table.reshape(-1)

    mesh = plsc.VectorSubcoreMesh(core_axis_name="c", subcore_axis_name="s")
    k = pl.kernel(
        functools.partial(_sc_body, bpw, nchunks, nc),
        out_type=jax.ShapeDtypeStruct((BATCH,), jnp.float32),
        mesh=mesh,
        compiler_params=pltpu.CompilerParams(needs_layout_passes=False,
                                             use_tc_tiling_on_sc=False),
        scratch_types=[
            pltpu.VMEM((nchunks, CHUNK), jnp.int32),
            pltpu.VMEM((nchunks, CHUNK), jnp.int32),
            pltpu.VMEM((bpw, EMBED), jnp.float32),
            pltpu.VMEM((bpw, EMBED), jnp.float32),
            pltpu.VMEM((bpw,), jnp.float32),
            pltpu.VMEM((bpw,), jnp.float32),
            pltpu.VMEM((bpw,), jnp.float32),
            pltpu.VMEM((LANES * LANES,), jnp.float32),
            pltpu.SemaphoreType.DMA,
        ],
    )
    return k(cidx, aidx, customer_emb_table, article_emb_table, cbias, abias)
